# baseline (speedup 1.0000x reference)
import jax
import jax.numpy as jnp
from jax.experimental import pallas as pl
from jax.experimental.pallas import tpu as pltpu

BETA = 0.1
VBLK = 65536
SPLIT = 24


def _add_kernel(s_ref, o_ref):
    o_ref[...] = s_ref[...] + jnp.asarray(1.0, s_ref.dtype)


def _part(scores, row0, nrows):
    vocab = scores.shape[1]
    nblk = pl.cdiv(vocab, VBLK)
    return pl.pallas_call(
        _add_kernel,
        out_shape=jax.ShapeDtypeStruct((nrows, vocab), scores.dtype),
        grid=(nblk,),
        in_specs=[pl.BlockSpec((nrows, VBLK), lambda j: (row0 // nrows, j))],
        out_specs=pl.BlockSpec((nrows, VBLK), lambda j: (0, j)),
        compiler_params=pltpu.CompilerParams(
            dimension_semantics=("arbitrary",),
        ),
    )(scores)


def kernel(input_ids, scores, cur_len, entity_token_ids):
    del input_ids, cur_len, entity_token_ids
    top = _part(scores, 0, SPLIT)
    bot = _part(scores, SPLIT, scores.shape[0] - SPLIT)
    return jnp.concatenate([top, bot], axis=0)


# speedup vs baseline: 1.2608x; 1.2608x over previous
import dataclasses
import functools

import jax
import jax.numpy as jnp
from jax.experimental import pallas as pl
from jax.experimental.pallas import tpu as pltpu
from jax.experimental.pallas import tpu_sc as plsc

BETA = 0.1
VBLK = 65536
N_WORKERS = 32
CHUNK = 31296


def _sc_boost_kernel(vocab, n_ent, ids_hbm, out_hbm, buf, ids_vmem, sem):
    w = jax.lax.axis_index("core") * 16 + jax.lax.axis_index("subcore")
    lo = w * CHUNK
    last = vocab - (N_WORKERS - 1) * CHUNK
    size = jnp.where(w == N_WORKERS - 1, last, CHUNK)
    hi = lo + size

    @pl.loop(0, CHUNK, step=16)
    def _(i):
        buf[pl.ds(i, 16)] = jnp.zeros((16,), buf.dtype)

    pltpu.async_copy(ids_hbm, ids_vmem, sem).wait()

    beta_vec = jnp.full((16,), BETA, buf.dtype)

    @pl.loop(0, n_ent, step=16)
    def _(i):
        idx = ids_vmem[pl.ds(i, 16)]
        m = (idx >= lo) & (idx < hi)
        loc = jnp.where(m, idx - lo, 0)
        plsc.store_scatter(buf, [loc], beta_vec, mask=m)

    @pl.when(w < N_WORKERS - 1)
    def _():
        pltpu.async_copy(buf.at[pl.ds(0, CHUNK)],
                         out_hbm.at[pl.ds(lo, CHUNK)], sem).wait()

    @pl.when(w == N_WORKERS - 1)
    def _():
        pltpu.async_copy(buf.at[pl.ds(0, last)],
                         out_hbm.at[pl.ds(lo, last)], sem).wait()


def _make_boost(entity_token_ids, vocab, dtype):
    n_ent = entity_token_ids.shape[0]
    cp = pltpu.CompilerParams()
    if "needs_layout_passes" in pltpu.CompilerParams.__dataclass_fields__:
        cp = dataclasses.replace(cp, needs_layout_passes=False)
    boost = pl.kernel(
        functools.partial(_sc_boost_kernel, vocab, n_ent),
        out_type=jax.ShapeDtypeStruct((vocab,), dtype),
        mesh=plsc.VectorSubcoreMesh(core_axis_name="core",
                                    subcore_axis_name="subcore"),
        scratch_types=[
            pltpu.VMEM((CHUNK,), dtype),
            pltpu.VMEM((n_ent,), jnp.int32),
            pltpu.SemaphoreType.DMA,
        ],
        compiler_params=cp,
    )(entity_token_ids)
    return boost


def _add_kernel(s_ref, b_ref, o_ref):
    o_ref[...] = s_ref[...] + b_ref[...]


def kernel(input_ids, scores, cur_len, entity_token_ids):
    del input_ids, cur_len
    batch, vocab = scores.shape

    boost = _make_boost(entity_token_ids.astype(jnp.int32), vocab,
                        scores.dtype).reshape(1, vocab)

    nblk = pl.cdiv(vocab, VBLK)
    out = pl.pallas_call(
        _add_kernel,
        out_shape=jax.ShapeDtypeStruct((batch, vocab), scores.dtype),
        grid=(nblk,),
        in_specs=[
            pl.BlockSpec((batch, VBLK), lambda j: (0, j)),
            pl.BlockSpec((1, VBLK), lambda j: (0, j)),
        ],
        out_specs=pl.BlockSpec((batch, VBLK), lambda j: (0, j)),
        compiler_params=pltpu.CompilerParams(
            dimension_semantics=("arbitrary",),
        ),
    )(scores, boost)
    return out


# speedup vs baseline: 1.9093x; 1.5143x over previous
import jax
import jax.numpy as jnp
from jax.experimental import pallas as pl
from jax.experimental.pallas import tpu as pltpu

BETA = 0.1
VBLK = 65536


def _fused_kernel(ids_ref, s_ref, o_ref, boost_ref, lists_ref, counts_ref):
    j = pl.program_id(0)
    nblk = pl.num_programs(0)
    n_ent = ids_ref.shape[0]

    @pl.when(j == 0)
    def _():
        def zero_counts(b, _):
            counts_ref[b] = 0
            return 0

        jax.lax.fori_loop(0, nblk, zero_counts, 0)

        def bin_one(i, _):
            e = ids_ref[i]
            blk = jax.lax.shift_right_logical(e, 16)
            c = counts_ref[blk]
            lists_ref[blk, c] = e
            counts_ref[blk] = c + 1
            return 0

        jax.lax.fori_loop(0, n_ent, bin_one, 0)

    boost_ref[...] = jnp.zeros_like(boost_ref)
    lane_iota = jax.lax.broadcasted_iota(jnp.int32, (1, 128), 1)
    blk_lo = j * VBLK

    def scatter_one(i, _):
        e = lists_ref[j, i] - blk_lo
        base = pl.multiple_of((e // 128) * 128, 128)
        row = boost_ref[0:1, pl.ds(base, 128)]
        row = jnp.where(lane_iota == e - base, jnp.asarray(BETA, row.dtype), row)
        boost_ref[0:1, pl.ds(base, 128)] = row
        return 0

    jax.lax.fori_loop(0, counts_ref[j], scatter_one, 0)

    o_ref[...] = s_ref[...] + boost_ref[...]


def kernel(input_ids, scores, cur_len, entity_token_ids):
    del input_ids, cur_len
    batch, vocab = scores.shape
    nblk = pl.cdiv(vocab, VBLK)
    n_ent = entity_token_ids.shape[0]

    return pl.pallas_call(
        _fused_kernel,
        out_shape=jax.ShapeDtypeStruct((batch, vocab), scores.dtype),
        grid=(nblk,),
        in_specs=[
            pl.BlockSpec(memory_space=pltpu.SMEM),
            pl.BlockSpec((batch, VBLK), lambda j: (0, j)),
        ],
        out_specs=pl.BlockSpec((batch, VBLK), lambda j: (0, j)),
        scratch_shapes=[
            pltpu.VMEM((1, VBLK), scores.dtype),
            pltpu.SMEM((nblk, n_ent), jnp.int32),
            pltpu.SMEM((nblk,), jnp.int32),
        ],
        compiler_params=pltpu.CompilerParams(
            dimension_semantics=("arbitrary",),
        ),
    )(entity_token_ids.astype(jnp.int32), scores)
